# Initial kernel scaffold; baseline (speedup 1.0000x reference)
import jax
import jax.numpy as jnp
from jax.experimental import pallas as pl


def kernel(x, edge_idx, W1, b1, W2, b2):
    raise NotImplementedError("write your pallas kernel here")


# speedup vs baseline: 13.9005x; 13.9005x over previous
import functools

import jax
import jax.numpy as jnp
import numpy as np
from jax import lax
from jax.experimental import pallas as pl
from jax.experimental.pallas import tpu as pltpu
from jax.experimental.pallas import tpu_sc as plsc

N_NODES = 10000
N_PAD = 10240
E = 160000
E_PAD = 163840
FH = 128
NHID = 512
NFEAT = 256

NSUB = 16
STRIPE = N_PAD // NSUB
ECHUNK = 128
KC = 8
ROWS_E = E_PAD // ECHUNK
CPS = ROWS_E // NSUB
CPT = ROWS_E // 32

_SC_CACHE = {}


def _sc_kernels():
    if _SC_CACHE:
        return _SC_CACHE["deg"], _SC_CACHE["agg"]

    mesh = plsc.VectorSubcoreMesh(core_axis_name="c", subcore_axis_name="s")

    @functools.partial(
        pl.kernel,
        out_type=jax.ShapeDtypeStruct((2, N_PAD, FH), jnp.float32),
        mesh=mesh,
        scratch_types=[
            pltpu.VMEM((CPT, ECHUNK), jnp.int32),
            pltpu.VMEM((ECHUNK, FH), jnp.float32),
            pltpu.VMEM_SHARED((N_PAD, FH), jnp.float32),
        ],
    )
    def _deg_kernel(dst_hbm, ones_hbm, zeros_hbm, out_hbm, idx_v, ones_v, acc_sh):
        c = lax.axis_index("c")
        s = lax.axis_index("s")
        wid = s * 2 + c
        r0 = s * STRIPE
        pltpu.sync_copy(ones_hbm, ones_v)
        pltpu.sync_copy(
            zeros_hbm.at[pl.ds(r0, STRIPE)], acc_sh.at[pl.ds(r0, STRIPE)]
        )
        pltpu.sync_copy(dst_hbm.at[pl.ds(wid * CPT, CPT)], idx_v)
        plsc.subcore_barrier()

        @pl.loop(0, CPT)
        def _(j):
            pltpu.sync_copy(ones_v, acc_sh.at[idx_v.at[j]], add=True)

        plsc.subcore_barrier()
        pltpu.sync_copy(
            acc_sh.at[pl.ds(r0, STRIPE)], out_hbm.at[c, pl.ds(r0, STRIPE)]
        )

    @functools.partial(
        pl.kernel,
        out_type=jax.ShapeDtypeStruct((2, N_PAD, FH), jnp.float32),
        mesh=mesh,
        scratch_types=[
            pltpu.VMEM((KC, ECHUNK), jnp.int32),
            pltpu.VMEM((KC, ECHUNK), jnp.int32),
            pltpu.VMEM((ECHUNK, FH), jnp.float32),
            pltpu.VMEM_SHARED((N_PAD, FH), jnp.float32),
        ],
    )
    def _agg_kernel(u_hbm, src_hbm, dst_hbm, out_hbm, src_v, dst_v, rows_v, acc_sh):
        c = lax.axis_index("c")
        s = lax.axis_index("s")
        r0 = s * STRIPE
        pltpu.sync_copy(
            u_hbm.at[pl.ds(c * N_PAD + r0, STRIPE)], acc_sh.at[pl.ds(r0, STRIPE)]
        )
        plsc.subcore_barrier()

        @pl.loop(0, CPS // KC)
        def _(i):
            row0 = s * CPS + i * KC
            pltpu.sync_copy(src_hbm.at[c, pl.ds(row0, KC)], src_v)
            pltpu.sync_copy(dst_hbm.at[pl.ds(row0, KC)], dst_v)

            @pl.loop(0, KC)
            def _(j):
                pltpu.sync_copy(u_hbm.at[src_v.at[j]], rows_v)
                pltpu.sync_copy(rows_v, acc_sh.at[dst_v.at[j]], add=True)

        plsc.subcore_barrier()
        pltpu.sync_copy(
            acc_sh.at[pl.ds(r0, STRIPE)], out_hbm.at[c, pl.ds(r0, STRIPE)]
        )

    _SC_CACHE["deg"] = _deg_kernel
    _SC_CACHE["agg"] = _agg_kernel
    return _deg_kernel, _agg_kernel


def _tc1_body(degp_ref, x_ref, u_ref, dinv_ref):
    deg = degp_ref[0, :, 0:1] + degp_ref[1, :, 0:1] + 1.0
    dinv = lax.rsqrt(deg)
    dinv_ref[...] = dinv
    u_ref[0, :, :] = x_ref[...] * dinv


def _tc1(degp, x_pad):
    return pl.pallas_call(
        _tc1_body,
        grid=(2,),
        in_specs=[
            pl.BlockSpec((2, N_PAD, FH), lambda j: (0, 0, 0)),
            pl.BlockSpec((N_PAD, FH), lambda j: (0, j)),
        ],
        out_specs=[
            pl.BlockSpec((1, N_PAD, FH), lambda j: (j, 0, 0)),
            pl.BlockSpec((N_PAD, 1), lambda j: (0, 0)),
        ],
        out_shape=[
            jax.ShapeDtypeStruct((2, N_PAD, FH), jnp.float32),
            jax.ShapeDtypeStruct((N_PAD, 1), jnp.float32),
        ],
    )(degp, x_pad)


R2 = 2560


def _tc2_body(w1_ref, dinv_ref, mask_ref, W1_ref, b1_ref, W2_ref, u2_ref):
    dinv = dinv_ref[...]
    a0 = w1_ref[0] * dinv
    a1 = w1_ref[1] * dinv
    a = jnp.concatenate([a0, a1], axis=1)
    h = (
        jnp.dot(a, W1_ref[...], preferred_element_type=jnp.float32,
                precision=lax.Precision.HIGHEST)
        + b1_ref[...]
    )
    h = jnp.where(h > 0, h, 0.01 * h)
    h = h * mask_ref[...]
    p = jnp.dot(h, W2_ref[...], preferred_element_type=jnp.float32,
                precision=lax.Precision.HIGHEST)
    u2_ref[0, :, :] = p[:, :FH] * dinv
    u2_ref[1, :, :] = p[:, FH:] * dinv


def _tc2(w1, dinv, maskf, W1, b1, W2):
    return pl.pallas_call(
        _tc2_body,
        grid=(N_PAD // R2,),
        in_specs=[
            pl.BlockSpec((2, R2, FH), lambda i: (0, i, 0)),
            pl.BlockSpec((R2, 1), lambda i: (i, 0)),
            pl.BlockSpec((R2, NHID), lambda i: (i, 0)),
            pl.BlockSpec((NFEAT, NHID), lambda i: (0, 0)),
            pl.BlockSpec((1, NHID), lambda i: (0, 0)),
            pl.BlockSpec((NHID, NFEAT), lambda i: (0, 0)),
        ],
        out_specs=pl.BlockSpec((2, R2, FH), lambda i: (0, i, 0)),
        out_shape=jax.ShapeDtypeStruct((2, N_PAD, FH), jnp.float32),
    )(w1, dinv, maskf, W1, b1, W2)


R3 = 2000


def _tc3_body(w2_ref, dinv_ref, b2_ref, o_ref):
    dinv = dinv_ref[...]
    a0 = w2_ref[0] * dinv
    a1 = w2_ref[1] * dinv
    o = jnp.concatenate([a0, a1], axis=1) + b2_ref[...]
    o_ref[...] = jnp.where(o > 0, o, 0.01 * o)


def _tc3(w2, dinv, b2):
    return pl.pallas_call(
        _tc3_body,
        grid=(N_NODES // R3,),
        in_specs=[
            pl.BlockSpec((2, R3, FH), lambda i: (0, i, 0)),
            pl.BlockSpec((R3, 1), lambda i: (i, 0)),
            pl.BlockSpec((1, NFEAT), lambda i: (0, 0)),
        ],
        out_specs=pl.BlockSpec((R3, NFEAT), lambda i: (i, 0)),
        out_shape=jax.ShapeDtypeStruct((N_NODES, NFEAT), jnp.float32),
    )(w2, dinv, b2)


_MASK_CACHE = []


def _np_rotl(x, d):
    return ((x << np.uint32(d)) | (x >> np.uint32(32 - d))).astype(np.uint32)


def _np_threefry2x32(k0, k1, x0, x1):
    x0 = x0.astype(np.uint32).copy()
    x1 = x1.astype(np.uint32).copy()
    ks = [np.uint32(k0), np.uint32(k1),
          np.uint32(np.uint32(k0) ^ np.uint32(k1) ^ np.uint32(0x1BD11BDA))]
    rots = [[13, 15, 26, 6], [17, 29, 16, 24]]
    x0 += ks[0]
    x1 += ks[1]
    for r in range(5):
        for d in rots[r % 2]:
            x0 += x1
            x1 = _np_rotl(x1, d)
            x1 ^= x0
        x0 += ks[(r + 1) % 3]
        x1 += ks[(r + 2) % 3] + np.uint32(r + 1)
    return x0, x1


def _dropout_mask():
    if not _MASK_CACHE:
        n = N_NODES * NHID
        b0, b1 = _np_threefry2x32(np.uint32(0), np.uint32(42),
                                  np.zeros(n, np.uint32),
                                  np.arange(n, dtype=np.uint32))
        bits = b0 ^ b1
        fl = ((bits >> np.uint32(9)) | np.uint32(0x3F800000)).view(np.float32)
        keep = (fl - 1.0) < 0.5
        mf = np.zeros((N_PAD, NHID), np.float32)
        mf[:N_NODES] = keep.reshape(N_NODES, NHID).astype(np.float32) * 2.0
        _MASK_CACHE.append(jnp.asarray(mf))
    return _MASK_CACHE[0]


def kernel(x, edge_idx, W1, b1, W2, b2):
    ei = edge_idx.astype(jnp.int32)
    src, dst = ei[0], ei[1]
    pad_n = E_PAD - E
    pad_idx = N_NODES + (jnp.arange(pad_n, dtype=jnp.int32) % (N_PAD - N_NODES))
    src_p = jnp.concatenate([src, pad_idx])
    dst_p3 = jnp.concatenate([dst, pad_idx]).reshape(ROWS_E, ECHUNK)
    src3 = jnp.stack([src_p, src_p + N_PAD]).reshape(2, ROWS_E, ECHUNK)
    x_pad = jnp.pad(x, ((0, N_PAD - N_NODES), (0, 0)))
    ones_rows = jnp.ones((ECHUNK, FH), jnp.float32)
    zeros_rows = jnp.zeros((N_PAD, FH), jnp.float32)

    deg_k, agg_k = _sc_kernels()
    degp = deg_k(dst_p3, ones_rows, zeros_rows)
    u1, dinv = _tc1(degp, x_pad)
    w1 = agg_k(u1.reshape(2 * N_PAD, FH), src3, dst_p3)
    u2 = _tc2(w1, dinv, _dropout_mask(), W1, b1.reshape(1, NHID), W2)
    w2 = agg_k(u2.reshape(2 * N_PAD, FH), src3, dst_p3)
    return _tc3(w2, dinv, b2.reshape(1, NFEAT))


# speedup vs baseline: 16.7583x; 1.2056x over previous
import functools

import jax
import jax.numpy as jnp
import numpy as np
from jax import lax
from jax.experimental import pallas as pl
from jax.experimental.pallas import tpu as pltpu
from jax.experimental.pallas import tpu_sc as plsc

N_NODES = 10000
N_PAD = 10240
E = 160000
E_PAD = 163840
FH = 128
NHID = 512
NFEAT = 256

NSUB = 16
STRIPE = N_PAD // NSUB
ECHUNK = 128
KC = 8
NBUF = 2
ROWS_E = E_PAD // ECHUNK
CPS = ROWS_E // NSUB
CPT = ROWS_E // 32

_SC_CACHE = {}


def _sc_kernels():
    if _SC_CACHE:
        return _SC_CACHE["deg"], _SC_CACHE["agg"]

    mesh = plsc.VectorSubcoreMesh(core_axis_name="c", subcore_axis_name="s")

    @functools.partial(
        pl.kernel,
        out_type=jax.ShapeDtypeStruct((2, N_PAD, FH), jnp.float32),
        mesh=mesh,
        scratch_types=[
            pltpu.VMEM((CPT, ECHUNK), jnp.int32),
            pltpu.VMEM((ECHUNK, FH), jnp.float32),
            pltpu.VMEM_SHARED((N_PAD, FH), jnp.float32),
        ],
    )
    def _deg_kernel(dst_hbm, ones_hbm, zeros_hbm, out_hbm, idx_v, ones_v, acc_sh):
        c = lax.axis_index("c")
        s = lax.axis_index("s")
        wid = s * 2 + c
        r0 = s * STRIPE
        pltpu.sync_copy(ones_hbm, ones_v)
        pltpu.sync_copy(
            zeros_hbm.at[pl.ds(r0, STRIPE)], acc_sh.at[pl.ds(r0, STRIPE)]
        )
        pltpu.sync_copy(dst_hbm.at[pl.ds(wid * CPT, CPT)], idx_v)
        plsc.subcore_barrier()

        @pl.loop(0, CPT)
        def _(j):
            pltpu.sync_copy(ones_v, acc_sh.at[idx_v.at[j]], add=True)

        plsc.subcore_barrier()
        pltpu.sync_copy(
            acc_sh.at[pl.ds(r0, STRIPE)], out_hbm.at[c, pl.ds(r0, STRIPE)]
        )

    @functools.partial(
        pl.kernel,
        out_type=jax.ShapeDtypeStruct((2, N_PAD, FH), jnp.float32),
        mesh=mesh,
        scratch_types=[
            pltpu.VMEM((KC, ECHUNK), jnp.int32),
            pltpu.VMEM((KC, ECHUNK), jnp.int32),
            pltpu.VMEM((NBUF, ECHUNK, FH), jnp.float32),
            pltpu.VMEM_SHARED((N_PAD, FH), jnp.float32),
        ],
    )
    def _agg_kernel(u_hbm, src_hbm, dst_hbm, out_hbm, src_v, dst_v, rows_v,
                    acc_sh):
        c = lax.axis_index("c")
        s = lax.axis_index("s")
        r0 = s * STRIPE
        pltpu.sync_copy(
            u_hbm.at[pl.ds(c * N_PAD + r0, STRIPE)], acc_sh.at[pl.ds(r0, STRIPE)]
        )
        plsc.subcore_barrier()

        def _edge_loop(gsem):
            @pl.loop(0, CPS // KC)
            def _(i):
                row0 = s * CPS + i * KC
                pltpu.sync_copy(src_hbm.at[c, pl.ds(row0, KC)], src_v)
                pltpu.sync_copy(dst_hbm.at[pl.ds(row0, KC)], dst_v)
                for j0 in range(0, KC, NBUF):
                    for b in range(NBUF):
                        pltpu.make_async_copy(
                            u_hbm.at[src_v.at[j0 + b]], rows_v.at[b], gsem
                        ).start()
                    for b in range(NBUF):
                        pltpu.make_async_copy(
                            u_hbm.at[src_v.at[j0 + b]], rows_v.at[b], gsem
                        ).wait()
                        pltpu.sync_copy(
                            rows_v.at[b], acc_sh.at[dst_v.at[j0 + b]], add=True
                        )

        pl.run_scoped(_edge_loop, gsem=pltpu.SemaphoreType.DMA)
        plsc.subcore_barrier()
        pltpu.sync_copy(
            acc_sh.at[pl.ds(r0, STRIPE)], out_hbm.at[c, pl.ds(r0, STRIPE)]
        )

    _SC_CACHE["deg"] = _deg_kernel
    _SC_CACHE["agg"] = _agg_kernel
    return _deg_kernel, _agg_kernel


def _tc1_body(degp_ref, x_ref, u_ref, dinv_ref):
    deg = degp_ref[0, :, 0:1] + degp_ref[1, :, 0:1] + 1.0
    dinv = lax.rsqrt(deg)
    dinv_ref[...] = dinv
    u_ref[0, :, :] = x_ref[...] * dinv


def _tc1(degp, x_pad):
    return pl.pallas_call(
        _tc1_body,
        grid=(2,),
        in_specs=[
            pl.BlockSpec((2, N_PAD, FH), lambda j: (0, 0, 0)),
            pl.BlockSpec((N_PAD, FH), lambda j: (0, j)),
        ],
        out_specs=[
            pl.BlockSpec((1, N_PAD, FH), lambda j: (j, 0, 0)),
            pl.BlockSpec((N_PAD, 1), lambda j: (0, 0)),
        ],
        out_shape=[
            jax.ShapeDtypeStruct((2, N_PAD, FH), jnp.float32),
            jax.ShapeDtypeStruct((N_PAD, 1), jnp.float32),
        ],
    )(degp, x_pad)


R2 = 2560


def _tc2_body(w1_ref, dinv_ref, mask_ref, W1_ref, b1_ref, W2_ref, u2_ref):
    dinv = dinv_ref[...]
    a0 = w1_ref[0] * dinv
    a1 = w1_ref[1] * dinv
    a = jnp.concatenate([a0, a1], axis=1)
    h = (
        jnp.dot(a, W1_ref[...], preferred_element_type=jnp.float32,
                precision=lax.Precision.HIGHEST)
        + b1_ref[...]
    )
    h = jnp.where(h > 0, h, 0.01 * h)
    h = h * mask_ref[...]
    p = jnp.dot(h, W2_ref[...], preferred_element_type=jnp.float32,
                precision=lax.Precision.HIGHEST)
    u2_ref[0, :, :] = p[:, :FH] * dinv
    u2_ref[1, :, :] = p[:, FH:] * dinv


def _tc2(w1, dinv, maskf, W1, b1, W2):
    return pl.pallas_call(
        _tc2_body,
        grid=(N_PAD // R2,),
        in_specs=[
            pl.BlockSpec((2, R2, FH), lambda i: (0, i, 0)),
            pl.BlockSpec((R2, 1), lambda i: (i, 0)),
            pl.BlockSpec((R2, NHID), lambda i: (i, 0)),
            pl.BlockSpec((NFEAT, NHID), lambda i: (0, 0)),
            pl.BlockSpec((1, NHID), lambda i: (0, 0)),
            pl.BlockSpec((NHID, NFEAT), lambda i: (0, 0)),
        ],
        out_specs=pl.BlockSpec((2, R2, FH), lambda i: (0, i, 0)),
        out_shape=jax.ShapeDtypeStruct((2, N_PAD, FH), jnp.float32),
    )(w1, dinv, maskf, W1, b1, W2)


R3 = 2000


def _tc3_body(w2_ref, dinv_ref, b2_ref, o_ref):
    dinv = dinv_ref[...]
    a0 = w2_ref[0] * dinv
    a1 = w2_ref[1] * dinv
    o = jnp.concatenate([a0, a1], axis=1) + b2_ref[...]
    o_ref[...] = jnp.where(o > 0, o, 0.01 * o)


def _tc3(w2, dinv, b2):
    return pl.pallas_call(
        _tc3_body,
        grid=(N_NODES // R3,),
        in_specs=[
            pl.BlockSpec((2, R3, FH), lambda i: (0, i, 0)),
            pl.BlockSpec((R3, 1), lambda i: (i, 0)),
            pl.BlockSpec((1, NFEAT), lambda i: (0, 0)),
        ],
        out_specs=pl.BlockSpec((R3, NFEAT), lambda i: (i, 0)),
        out_shape=jax.ShapeDtypeStruct((N_NODES, NFEAT), jnp.float32),
    )(w2, dinv, b2)


_MASK_CACHE = []


def _np_rotl(x, d):
    return ((x << np.uint32(d)) | (x >> np.uint32(32 - d))).astype(np.uint32)


def _np_threefry2x32(k0, k1, x0, x1):
    x0 = x0.astype(np.uint32).copy()
    x1 = x1.astype(np.uint32).copy()
    ks = [np.uint32(k0), np.uint32(k1),
          np.uint32(np.uint32(k0) ^ np.uint32(k1) ^ np.uint32(0x1BD11BDA))]
    rots = [[13, 15, 26, 6], [17, 29, 16, 24]]
    x0 += ks[0]
    x1 += ks[1]
    for r in range(5):
        for d in rots[r % 2]:
            x0 += x1
            x1 = _np_rotl(x1, d)
            x1 ^= x0
        x0 += ks[(r + 1) % 3]
        x1 += ks[(r + 2) % 3] + np.uint32(r + 1)
    return x0, x1


def _dropout_mask():
    if not _MASK_CACHE:
        n = N_NODES * NHID
        b0, b1 = _np_threefry2x32(np.uint32(0), np.uint32(42),
                                  np.zeros(n, np.uint32),
                                  np.arange(n, dtype=np.uint32))
        bits = b0 ^ b1
        fl = ((bits >> np.uint32(9)) | np.uint32(0x3F800000)).view(np.float32)
        keep = (fl - 1.0) < 0.5
        mf = np.zeros((N_PAD, NHID), np.float32)
        mf[:N_NODES] = keep.reshape(N_NODES, NHID).astype(np.float32) * 2.0
        _MASK_CACHE.append(jnp.asarray(mf))
    return _MASK_CACHE[0]


def kernel(x, edge_idx, W1, b1, W2, b2):
    ei = edge_idx.astype(jnp.int32)
    src, dst = ei[0], ei[1]
    pad_n = E_PAD - E
    pad_idx = N_NODES + (jnp.arange(pad_n, dtype=jnp.int32) % (N_PAD - N_NODES))
    src_p = jnp.concatenate([src, pad_idx])
    dst_p3 = jnp.concatenate([dst, pad_idx]).reshape(ROWS_E, ECHUNK)
    src3 = jnp.stack([src_p, src_p + N_PAD]).reshape(2, ROWS_E, ECHUNK)
    x_pad = jnp.pad(x, ((0, N_PAD - N_NODES), (0, 0)))
    ones_rows = jnp.ones((ECHUNK, FH), jnp.float32)
    zeros_rows = jnp.zeros((N_PAD, FH), jnp.float32)

    deg_k, agg_k = _sc_kernels()
    degp = deg_k(dst_p3, ones_rows, zeros_rows)
    u1, dinv = _tc1(degp, x_pad)
    w1 = agg_k(u1.reshape(2 * N_PAD, FH), src3, dst_p3)
    u2 = _tc2(w1, dinv, _dropout_mask(), W1, b1.reshape(1, NHID), W2)
    w2 = agg_k(u2.reshape(2 * N_PAD, FH), src3, dst_p3)
    return _tc3(w2, dinv, b2.reshape(1, NFEAT))


# speedup vs baseline: 17.5584x; 1.0477x over previous
import functools

import jax
import jax.numpy as jnp
import numpy as np
from jax import lax
from jax.experimental import pallas as pl
from jax.experimental.pallas import tpu as pltpu
from jax.experimental.pallas import tpu_sc as plsc

N_NODES = 10000
N_PAD = 10240
E = 160000
E_PAD = 163840
FH = 128
NHID = 512
NFEAT = 256

NSUB = 16
STRIPE = N_PAD // NSUB
ECHUNK = 128
KC = 8
HKC = 40
NBUF = 2
ROWS_E = E_PAD // ECHUNK
CPS = ROWS_E // NSUB
CPT = ROWS_E // 32

_SC_CACHE = {}


def _sc_kernels():
    if _SC_CACHE:
        return _SC_CACHE["deg"], _SC_CACHE["agg"]

    mesh = plsc.VectorSubcoreMesh(core_axis_name="c", subcore_axis_name="s")

    @functools.partial(
        pl.kernel,
        out_type=jax.ShapeDtypeStruct((2, N_PAD, FH), jnp.float32),
        mesh=mesh,
        scratch_types=[
            pltpu.VMEM((CPT, ECHUNK), jnp.int32),
            pltpu.VMEM((ECHUNK, FH), jnp.float32),
            pltpu.VMEM_SHARED((N_PAD, FH), jnp.float32),
        ],
    )
    def _deg_kernel(dst_hbm, ones_hbm, zeros_hbm, out_hbm, idx_v, ones_v, acc_sh):
        c = lax.axis_index("c")
        s = lax.axis_index("s")
        wid = s * 2 + c
        r0 = s * STRIPE
        pltpu.sync_copy(ones_hbm, ones_v)
        pltpu.sync_copy(
            zeros_hbm.at[pl.ds(r0, STRIPE)], acc_sh.at[pl.ds(r0, STRIPE)]
        )
        pltpu.sync_copy(dst_hbm.at[pl.ds(wid * CPT, CPT)], idx_v)
        plsc.subcore_barrier()

        def _scatter_all(ssem):
            @pl.loop(0, CPT)
            def _(j):
                pltpu.make_async_copy(
                    ones_v, acc_sh.at[idx_v.at[j]], ssem
                ).start(add=True)

            @pl.loop(0, CPT)
            def _(j):
                pltpu.make_async_copy(
                    ones_v, acc_sh.at[idx_v.at[j]], ssem
                ).wait()

        pl.run_scoped(_scatter_all, ssem=pltpu.SemaphoreType.DMA)
        plsc.subcore_barrier()
        pltpu.sync_copy(
            acc_sh.at[pl.ds(r0, STRIPE)], out_hbm.at[c, pl.ds(r0, STRIPE)]
        )

    @functools.partial(
        pl.kernel,
        out_type=jax.ShapeDtypeStruct((2, N_PAD, FH), jnp.float32),
        mesh=mesh,
        scratch_types=[
            pltpu.VMEM((HKC, ECHUNK), jnp.int32),
            pltpu.VMEM((HKC, ECHUNK), jnp.int32),
            pltpu.VMEM((NBUF, ECHUNK, FH), jnp.float32),
            pltpu.VMEM_SHARED((N_PAD, FH), jnp.float32),
        ],
    )
    def _agg_kernel(u_hbm, src_hbm, dst_hbm, out_hbm, src_v, dst_v, rows_v,
                    acc_sh):
        c = lax.axis_index("c")
        s = lax.axis_index("s")
        r0 = s * STRIPE
        pltpu.sync_copy(
            u_hbm.at[pl.ds(c * N_PAD + r0, STRIPE)], acc_sh.at[pl.ds(r0, STRIPE)]
        )
        plsc.subcore_barrier()

        def _edge_loop(gsem, ssem):
            def g_copy(k, b):
                return pltpu.make_async_copy(
                    u_hbm.at[src_v.at[k]], rows_v.at[b], gsem
                )

            def s_copy(k, b):
                return pltpu.make_async_copy(
                    rows_v.at[b], acc_sh.at[dst_v.at[k]], ssem
                )

            for h in range(2):
                row0 = s * CPS + h * HKC
                pltpu.sync_copy(src_hbm.at[c, pl.ds(row0, HKC)], src_v)
                pltpu.sync_copy(dst_hbm.at[pl.ds(row0, HKC)], dst_v)

                g_copy(0, 0).start()
                g_copy(1, 1).start()
                g_copy(0, 0).wait()
                s_copy(0, 0).start(add=True)
                g_copy(1, 1).wait()
                s_copy(1, 1).start(add=True)

                @pl.loop(1, HKC // 2)
                def _(i):
                    k0 = i * 2
                    for j in range(2):
                        k = k0 + j
                        s_copy(k - 2, j).wait()
                        g_copy(k, j).start()
                        g_copy(k, j).wait()
                        s_copy(k, j).start(add=True)

                s_copy(HKC - 2, 0).wait()
                s_copy(HKC - 1, 1).wait()

        pl.run_scoped(
            _edge_loop, gsem=pltpu.SemaphoreType.DMA, ssem=pltpu.SemaphoreType.DMA
        )
        plsc.subcore_barrier()
        pltpu.sync_copy(
            acc_sh.at[pl.ds(r0, STRIPE)], out_hbm.at[c, pl.ds(r0, STRIPE)]
        )

    _SC_CACHE["deg"] = _deg_kernel
    _SC_CACHE["agg"] = _agg_kernel
    return _deg_kernel, _agg_kernel


def _tc1_body(degp_ref, x_ref, u_ref, dinv_ref):
    deg = degp_ref[0, :, 0:1] + degp_ref[1, :, 0:1] + 1.0
    dinv = lax.rsqrt(deg)
    dinv_ref[...] = dinv
    u_ref[0, :, :] = x_ref[...] * dinv


def _tc1(degp, x_pad):
    return pl.pallas_call(
        _tc1_body,
        grid=(2,),
        in_specs=[
            pl.BlockSpec((2, N_PAD, FH), lambda j: (0, 0, 0)),
            pl.BlockSpec((N_PAD, FH), lambda j: (0, j)),
        ],
        out_specs=[
            pl.BlockSpec((1, N_PAD, FH), lambda j: (j, 0, 0)),
            pl.BlockSpec((N_PAD, 1), lambda j: (0, 0)),
        ],
        out_shape=[
            jax.ShapeDtypeStruct((2, N_PAD, FH), jnp.float32),
            jax.ShapeDtypeStruct((N_PAD, 1), jnp.float32),
        ],
    )(degp, x_pad)


R2 = 2560


def _tc2_body(w1_ref, dinv_ref, mask_ref, W1_ref, b1_ref, W2_ref, u2_ref):
    dinv = dinv_ref[...]
    a0 = w1_ref[0] * dinv
    a1 = w1_ref[1] * dinv
    a = jnp.concatenate([a0, a1], axis=1)
    h = (
        jnp.dot(a, W1_ref[...], preferred_element_type=jnp.float32,
                precision=lax.Precision.HIGHEST)
        + b1_ref[...]
    )
    h = jnp.where(h > 0, h, 0.01 * h)
    h = h * mask_ref[...]
    p = jnp.dot(h, W2_ref[...], preferred_element_type=jnp.float32,
                precision=lax.Precision.HIGHEST)
    u2_ref[0, :, :] = p[:, :FH] * dinv
    u2_ref[1, :, :] = p[:, FH:] * dinv


def _tc2(w1, dinv, maskf, W1, b1, W2):
    return pl.pallas_call(
        _tc2_body,
        grid=(N_PAD // R2,),
        in_specs=[
            pl.BlockSpec((2, R2, FH), lambda i: (0, i, 0)),
            pl.BlockSpec((R2, 1), lambda i: (i, 0)),
            pl.BlockSpec((R2, NHID), lambda i: (i, 0)),
            pl.BlockSpec((NFEAT, NHID), lambda i: (0, 0)),
            pl.BlockSpec((1, NHID), lambda i: (0, 0)),
            pl.BlockSpec((NHID, NFEAT), lambda i: (0, 0)),
        ],
        out_specs=pl.BlockSpec((2, R2, FH), lambda i: (0, i, 0)),
        out_shape=jax.ShapeDtypeStruct((2, N_PAD, FH), jnp.float32),
    )(w1, dinv, maskf, W1, b1, W2)


R3 = 2000


def _tc3_body(w2_ref, dinv_ref, b2_ref, o_ref):
    dinv = dinv_ref[...]
    a0 = w2_ref[0] * dinv
    a1 = w2_ref[1] * dinv
    o = jnp.concatenate([a0, a1], axis=1) + b2_ref[...]
    o_ref[...] = jnp.where(o > 0, o, 0.01 * o)


def _tc3(w2, dinv, b2):
    return pl.pallas_call(
        _tc3_body,
        grid=(N_NODES // R3,),
        in_specs=[
            pl.BlockSpec((2, R3, FH), lambda i: (0, i, 0)),
            pl.BlockSpec((R3, 1), lambda i: (i, 0)),
            pl.BlockSpec((1, NFEAT), lambda i: (0, 0)),
        ],
        out_specs=pl.BlockSpec((R3, NFEAT), lambda i: (i, 0)),
        out_shape=jax.ShapeDtypeStruct((N_NODES, NFEAT), jnp.float32),
    )(w2, dinv, b2)


_MASK_CACHE = []


def _np_rotl(x, d):
    return ((x << np.uint32(d)) | (x >> np.uint32(32 - d))).astype(np.uint32)


def _np_threefry2x32(k0, k1, x0, x1):
    x0 = x0.astype(np.uint32).copy()
    x1 = x1.astype(np.uint32).copy()
    ks = [np.uint32(k0), np.uint32(k1),
          np.uint32(np.uint32(k0) ^ np.uint32(k1) ^ np.uint32(0x1BD11BDA))]
    rots = [[13, 15, 26, 6], [17, 29, 16, 24]]
    x0 += ks[0]
    x1 += ks[1]
    for r in range(5):
        for d in rots[r % 2]:
            x0 += x1
            x1 = _np_rotl(x1, d)
            x1 ^= x0
        x0 += ks[(r + 1) % 3]
        x1 += ks[(r + 2) % 3] + np.uint32(r + 1)
    return x0, x1


def _dropout_mask():
    if not _MASK_CACHE:
        n = N_NODES * NHID
        b0, b1 = _np_threefry2x32(np.uint32(0), np.uint32(42),
                                  np.zeros(n, np.uint32),
                                  np.arange(n, dtype=np.uint32))
        bits = b0 ^ b1
        fl = ((bits >> np.uint32(9)) | np.uint32(0x3F800000)).view(np.float32)
        keep = (fl - 1.0) < 0.5
        mf = np.zeros((N_PAD, NHID), np.float32)
        mf[:N_NODES] = keep.reshape(N_NODES, NHID).astype(np.float32) * 2.0
        _MASK_CACHE.append(jnp.asarray(mf))
    return _MASK_CACHE[0]


def kernel(x, edge_idx, W1, b1, W2, b2):
    ei = edge_idx.astype(jnp.int32)
    src, dst = ei[0], ei[1]
    pad_n = E_PAD - E
    pad_idx = N_NODES + (jnp.arange(pad_n, dtype=jnp.int32) % (N_PAD - N_NODES))
    src_p = jnp.concatenate([src, pad_idx])
    dst_p3 = jnp.concatenate([dst, pad_idx]).reshape(ROWS_E, ECHUNK)
    src3 = jnp.stack([src_p, src_p + N_PAD]).reshape(2, ROWS_E, ECHUNK)
    x_pad = jnp.pad(x, ((0, N_PAD - N_NODES), (0, 0)))
    ones_rows = jnp.ones((ECHUNK, FH), jnp.float32)
    zeros_rows = jnp.zeros((N_PAD, FH), jnp.float32)

    deg_k, agg_k = _sc_kernels()
    degp = deg_k(dst_p3, ones_rows, zeros_rows)
    u1, dinv = _tc1(degp, x_pad)
    w1 = agg_k(u1.reshape(2 * N_PAD, FH), src3, dst_p3)
    u2 = _tc2(w1, dinv, _dropout_mask(), W1, b1.reshape(1, NHID), W2)
    w2 = agg_k(u2.reshape(2 * N_PAD, FH), src3, dst_p3)
    return _tc3(w2, dinv, b2.reshape(1, NFEAT))


# speedup vs baseline: 19.4124x; 1.1056x over previous
import functools

import jax
import jax.numpy as jnp
import numpy as np
from jax import lax
from jax.experimental import pallas as pl
from jax.experimental.pallas import tpu as pltpu
from jax.experimental.pallas import tpu_sc as plsc

N_NODES = 10000
N_PAD = 10240
E = 160000
E_PAD = 163840
FH = 128
NHID = 512
NFEAT = 256

NSUB = 16
STRIPE = N_PAD // NSUB
ECHUNK = 128
KC = 8
HKC = 40
NBUF = 2
ROWS_E = E_PAD // ECHUNK
CPS = ROWS_E // NSUB
CPT = ROWS_E // 32

_SC_CACHE = {}


def _sc_kernels():
    if _SC_CACHE:
        return _SC_CACHE["deg"], _SC_CACHE["agg"]

    mesh = plsc.VectorSubcoreMesh(core_axis_name="c", subcore_axis_name="s")

    @functools.partial(
        pl.kernel,
        out_type=jax.ShapeDtypeStruct((2, N_PAD, FH), jnp.float32),
        mesh=mesh,
        scratch_types=[
            pltpu.VMEM((CPT, ECHUNK), jnp.int32),
            pltpu.VMEM((ECHUNK, FH), jnp.float32),
            pltpu.VMEM_SHARED((N_PAD, FH), jnp.float32),
        ],
    )
    def _deg_kernel(dst_hbm, ones_hbm, zeros_hbm, out_hbm, idx_v, ones_v, acc_sh):
        c = lax.axis_index("c")
        s = lax.axis_index("s")
        wid = s * 2 + c
        r0 = s * STRIPE
        pltpu.sync_copy(ones_hbm, ones_v)
        pltpu.sync_copy(
            zeros_hbm.at[pl.ds(r0, STRIPE)], acc_sh.at[pl.ds(r0, STRIPE)]
        )
        pltpu.sync_copy(dst_hbm.at[pl.ds(wid * CPT, CPT)], idx_v)
        plsc.subcore_barrier()

        def _scatter_all(ssem):
            @pl.loop(0, CPT)
            def _(j):
                pltpu.make_async_copy(
                    ones_v, acc_sh.at[idx_v.at[j]], ssem
                ).start(add=True)

            @pl.loop(0, CPT)
            def _(j):
                pltpu.make_async_copy(
                    ones_v, acc_sh.at[idx_v.at[j]], ssem
                ).wait()

        pl.run_scoped(_scatter_all, ssem=pltpu.SemaphoreType.DMA)
        plsc.subcore_barrier()
        pltpu.sync_copy(
            acc_sh.at[pl.ds(r0, STRIPE)], out_hbm.at[c, pl.ds(r0, STRIPE)]
        )

    @functools.partial(
        pl.kernel,
        out_type=jax.ShapeDtypeStruct((2, N_PAD, FH), jnp.float32),
        mesh=mesh,
        scratch_types=[
            pltpu.VMEM((HKC, ECHUNK), jnp.int32),
            pltpu.VMEM((HKC, ECHUNK), jnp.int32),
            pltpu.VMEM((NBUF, ECHUNK, FH), jnp.float32),
            pltpu.VMEM_SHARED((N_PAD, FH), jnp.float32),
        ],
    )
    def _agg_kernel(u_hbm, src_hbm, dst_hbm, out_hbm, src_v, dst_v, rows_v,
                    acc_sh):
        c = lax.axis_index("c")
        s = lax.axis_index("s")
        r0 = s * STRIPE
        pltpu.sync_copy(
            u_hbm.at[pl.ds(c * N_PAD + r0, STRIPE)], acc_sh.at[pl.ds(r0, STRIPE)]
        )
        plsc.subcore_barrier()

        def _edge_loop(gsem, ssem):
            def g_copy(k, b):
                return pltpu.make_async_copy(
                    u_hbm.at[src_v.at[k]], rows_v.at[b], gsem
                )

            def s_copy(k, b):
                return pltpu.make_async_copy(
                    rows_v.at[b], acc_sh.at[dst_v.at[k]], ssem
                )

            for h in range(2):
                row0 = s * CPS + h * HKC
                pltpu.sync_copy(src_hbm.at[c, pl.ds(row0, HKC)], src_v)
                pltpu.sync_copy(dst_hbm.at[pl.ds(row0, HKC)], dst_v)

                g_copy(0, 0).start()
                g_copy(1, 1).start()
                g_copy(0, 0).wait()
                s_copy(0, 0).start(add=True)
                g_copy(1, 1).wait()
                s_copy(1, 1).start(add=True)

                @pl.loop(1, HKC // 2)
                def _(i):
                    k0 = i * 2
                    for j in range(2):
                        k = k0 + j
                        s_copy(k - 2, j).wait()
                        g_copy(k, j).start()
                        g_copy(k, j).wait()
                        s_copy(k, j).start(add=True)

                s_copy(HKC - 2, 0).wait()
                s_copy(HKC - 1, 1).wait()

        pl.run_scoped(
            _edge_loop, gsem=pltpu.SemaphoreType.DMA, ssem=pltpu.SemaphoreType.DMA
        )
        plsc.subcore_barrier()
        pltpu.sync_copy(
            acc_sh.at[pl.ds(r0, STRIPE)], out_hbm.at[c, pl.ds(r0, STRIPE)]
        )

    _SC_CACHE["deg"] = _deg_kernel
    _SC_CACHE["agg"] = _agg_kernel
    return _deg_kernel, _agg_kernel


def _tc1_body(degp_ref, x_ref, u_ref, dinv_ref):
    deg = degp_ref[0, :, 0:1] + degp_ref[1, :, 0:1] + 1.0
    dinv = lax.rsqrt(deg)
    dinv_ref[...] = dinv
    u_ref[0, :, :] = x_ref[...] * dinv


def _tc1(degp, x_pad):
    return pl.pallas_call(
        _tc1_body,
        grid=(2,),
        in_specs=[
            pl.BlockSpec((2, N_PAD, FH), lambda j: (0, 0, 0)),
            pl.BlockSpec((N_PAD, FH), lambda j: (0, j)),
        ],
        out_specs=[
            pl.BlockSpec((1, N_PAD, FH), lambda j: (j, 0, 0)),
            pl.BlockSpec((N_PAD, 1), lambda j: (0, 0)),
        ],
        out_shape=[
            jax.ShapeDtypeStruct((2, N_PAD, FH), jnp.float32),
            jax.ShapeDtypeStruct((N_PAD, 1), jnp.float32),
        ],
    )(degp, x_pad)


R2 = 2560


def _tc2_body(w1_ref, dinv_ref, mask_ref, W1_ref, b1_ref, W2_ref, u2_ref):
    dinv = dinv_ref[...]
    a0 = w1_ref[0] * dinv
    a1 = w1_ref[1] * dinv
    a = jnp.concatenate([a0, a1], axis=1)
    h = (
        jnp.dot(a, W1_ref[...], preferred_element_type=jnp.float32)
        + b1_ref[...]
    )
    h = jnp.where(h > 0, h, 0.01 * h)
    h = h * mask_ref[...].astype(jnp.float32)
    p = jnp.dot(h, W2_ref[...], preferred_element_type=jnp.float32)
    u2_ref[0, :, :] = p[:, :FH] * dinv
    u2_ref[1, :, :] = p[:, FH:] * dinv


def _tc2(w1, dinv, maskf, W1, b1, W2):
    return pl.pallas_call(
        _tc2_body,
        grid=(N_PAD // R2,),
        in_specs=[
            pl.BlockSpec((2, R2, FH), lambda i: (0, i, 0)),
            pl.BlockSpec((R2, 1), lambda i: (i, 0)),
            pl.BlockSpec((R2, NHID), lambda i: (i, 0)),
            pl.BlockSpec((NFEAT, NHID), lambda i: (0, 0)),
            pl.BlockSpec((1, NHID), lambda i: (0, 0)),
            pl.BlockSpec((NHID, NFEAT), lambda i: (0, 0)),
        ],
        out_specs=pl.BlockSpec((2, R2, FH), lambda i: (0, i, 0)),
        out_shape=jax.ShapeDtypeStruct((2, N_PAD, FH), jnp.float32),
    )(w1, dinv, maskf, W1, b1, W2)


R3 = 2000


def _tc3_body(w2_ref, dinv_ref, b2_ref, o_ref):
    dinv = dinv_ref[...]
    a0 = w2_ref[0] * dinv
    a1 = w2_ref[1] * dinv
    o = jnp.concatenate([a0, a1], axis=1) + b2_ref[...]
    o_ref[...] = jnp.where(o > 0, o, 0.01 * o)


def _tc3(w2, dinv, b2):
    return pl.pallas_call(
        _tc3_body,
        grid=(N_NODES // R3,),
        in_specs=[
            pl.BlockSpec((2, R3, FH), lambda i: (0, i, 0)),
            pl.BlockSpec((R3, 1), lambda i: (i, 0)),
            pl.BlockSpec((1, NFEAT), lambda i: (0, 0)),
        ],
        out_specs=pl.BlockSpec((R3, NFEAT), lambda i: (i, 0)),
        out_shape=jax.ShapeDtypeStruct((N_NODES, NFEAT), jnp.float32),
    )(w2, dinv, b2)


_MASK_CACHE = []


def _np_rotl(x, d):
    return ((x << np.uint32(d)) | (x >> np.uint32(32 - d))).astype(np.uint32)


def _np_threefry2x32(k0, k1, x0, x1):
    x0 = x0.astype(np.uint32).copy()
    x1 = x1.astype(np.uint32).copy()
    ks = [np.uint32(k0), np.uint32(k1),
          np.uint32(np.uint32(k0) ^ np.uint32(k1) ^ np.uint32(0x1BD11BDA))]
    rots = [[13, 15, 26, 6], [17, 29, 16, 24]]
    x0 += ks[0]
    x1 += ks[1]
    for r in range(5):
        for d in rots[r % 2]:
            x0 += x1
            x1 = _np_rotl(x1, d)
            x1 ^= x0
        x0 += ks[(r + 1) % 3]
        x1 += ks[(r + 2) % 3] + np.uint32(r + 1)
    return x0, x1


def _dropout_mask():
    if not _MASK_CACHE:
        n = N_NODES * NHID
        b0, b1 = _np_threefry2x32(np.uint32(0), np.uint32(42),
                                  np.zeros(n, np.uint32),
                                  np.arange(n, dtype=np.uint32))
        bits = b0 ^ b1
        fl = ((bits >> np.uint32(9)) | np.uint32(0x3F800000)).view(np.float32)
        keep = (fl - 1.0) < 0.5
        mf = np.zeros((N_PAD, NHID), np.float32)
        mf[:N_NODES] = keep.reshape(N_NODES, NHID).astype(np.float32) * 2.0
        _MASK_CACHE.append(jnp.asarray(mf, dtype=jnp.bfloat16))
    return _MASK_CACHE[0]


def kernel(x, edge_idx, W1, b1, W2, b2):
    ei = edge_idx.astype(jnp.int32)
    src, dst = ei[0], ei[1]
    pad_n = E_PAD - E
    pad_idx = N_NODES + (jnp.arange(pad_n, dtype=jnp.int32) % (N_PAD - N_NODES))
    src_p = jnp.concatenate([src, pad_idx])
    dst_p3 = jnp.concatenate([dst, pad_idx]).reshape(ROWS_E, ECHUNK)
    src3 = jnp.stack([src_p, src_p + N_PAD]).reshape(2, ROWS_E, ECHUNK)
    x_pad = jnp.pad(x, ((0, N_PAD - N_NODES), (0, 0)))
    ones_rows = jnp.ones((ECHUNK, FH), jnp.float32)
    zeros_rows = jnp.zeros((N_PAD, FH), jnp.float32)

    deg_k, agg_k = _sc_kernels()
    degp = deg_k(dst_p3, ones_rows, zeros_rows)
    u1, dinv = _tc1(degp, x_pad)
    w1 = agg_k(u1.reshape(2 * N_PAD, FH), src3, dst_p3)
    u2 = _tc2(w1, dinv, _dropout_mask(), W1, b1.reshape(1, NHID), W2)
    w2 = agg_k(u2.reshape(2 * N_PAD, FH), src3, dst_p3)
    return _tc3(w2, dinv, b2.reshape(1, NFEAT))
